# baseline (speedup 1.0000x reference)
import functools

import jax
import jax.numpy as jnp
from jax import lax
from jax.experimental import pallas as pl
from jax.experimental.pallas import tpu as pltpu
from jax.experimental.pallas import tpu_sc as plsc

D_MODEL = 64
SCALE = 8.0
NC, NS = 2, 16
NW = NC * NS
C = 128


def _build_sc_kernel(R):
    rows_per_w = R // NW
    chunks_per_w = rows_per_w // C
    mesh = plsc.VectorSubcoreMesh(core_axis_name="c", subcore_axis_name="s")

    @functools.partial(
        pl.kernel,
        out_type=jax.ShapeDtypeStruct((R, D_MODEL), jnp.float32),
        mesh=mesh,
        compiler_params=pltpu.CompilerParams(use_tc_tiling_on_sc=False),
        scratch_types=[
            pltpu.VMEM((chunks_per_w, C), jnp.int32),
            pltpu.VMEM((C, D_MODEL), jnp.float32),
            pltpu.SemaphoreType.DMA,
        ],
    )
    def k(lut_hbm, idx_hbm, out_hbm, idx_v, buf, gsem):
        wid = lax.axis_index("s") * NC + lax.axis_index("c")
        chunk0 = wid * chunks_per_w
        pltpu.sync_copy(idx_hbm.at[pl.ds(chunk0, chunks_per_w)], idx_v)

        def chunk_body(j, carry):
            pltpu.async_copy(lut_hbm.at[idx_v.at[j]], buf, gsem).wait()

            def row_body(r, c2):
                for d in range(D_MODEL // 16):
                    sl = pl.ds(d * 16, 16)
                    buf[r, sl] = buf[r, sl] * SCALE
                return c2

            lax.fori_loop(0, C, row_body, 0)
            pltpu.sync_copy(buf, out_hbm.at[pl.ds((chunk0 + j) * C, C)])
            return carry

        lax.fori_loop(0, chunks_per_w, chunk_body, 0)

    return k


def kernel(x, lut):
    B, S = x.shape
    R = B * S
    xt = jnp.transpose(x).reshape(R // C, C)
    out_flat = _build_sc_kernel(R)(lut, xt)
    return out_flat.reshape(S, B, D_MODEL)


# speedup vs baseline: 1.1974x; 1.1974x over previous
import functools

import jax
import jax.numpy as jnp
from jax import lax
from jax.experimental import pallas as pl
from jax.experimental.pallas import tpu as pltpu
from jax.experimental.pallas import tpu_sc as plsc

D_MODEL = 64
SCALE = 8.0
NC, NS = 2, 16
NW = NC * NS
C = 128


def _build_sc_kernel(R):
    rows_per_w = R // NW
    T = rows_per_w // C
    assert T % 4 == 0 and T >= 8
    groups = T // 4
    mesh = plsc.VectorSubcoreMesh(core_axis_name="c", subcore_axis_name="s")

    @functools.partial(
        pl.kernel,
        out_type=jax.ShapeDtypeStruct((R, D_MODEL), jnp.float32),
        mesh=mesh,
        compiler_params=pltpu.CompilerParams(use_tc_tiling_on_sc=False),
        scratch_types=[
            pltpu.VMEM((T, C), jnp.int32),
            pltpu.VMEM((C, D_MODEL), jnp.float32),
            pltpu.VMEM((C, D_MODEL), jnp.float32),
            pltpu.VMEM((C, D_MODEL), jnp.float32),
            pltpu.VMEM((C, D_MODEL), jnp.float32),
            pltpu.SemaphoreType.DMA,
            pltpu.SemaphoreType.DMA,
            pltpu.SemaphoreType.DMA,
            pltpu.SemaphoreType.DMA,
            pltpu.SemaphoreType.DMA,
            pltpu.SemaphoreType.DMA,
            pltpu.SemaphoreType.DMA,
            pltpu.SemaphoreType.DMA,
        ],
    )
    def k(lut_hbm, idx_hbm, out_hbm, idx_v,
          b0, b1, b2, b3, g0, g1, g2, g3, s0, s1, s2, s3):
        bufs = (b0, b1, b2, b3)
        gsem = (g0, g1, g2, g3)
        ssem = (s0, s1, s2, s3)
        wid = lax.axis_index("s") * NC + lax.axis_index("c")
        chunk0 = wid * T
        pltpu.sync_copy(idx_hbm.at[pl.ds(chunk0, T)], idx_v)

        def start_gather(j, slot):
            pltpu.async_copy(lut_hbm.at[idx_v.at[j]], bufs[slot], gsem[slot])

        def wait_gather(j, slot):
            pltpu.make_async_copy(
                lut_hbm.at[idx_v.at[j]], bufs[slot], gsem[slot]).wait()

        def scale(slot):
            buf = bufs[slot]

            @plsc.parallel_loop(0, C, 1, unroll=2)
            def _(r):
                for d in range(D_MODEL // 16):
                    sl = pl.ds(d * 16, 16)
                    buf[r, sl] = buf[r, sl] * SCALE

        def out_slice(j):
            return out_hbm.at[pl.ds((chunk0 + j) * C, C)]

        def start_scatter(j, slot):
            pltpu.async_copy(bufs[slot], out_slice(j), ssem[slot])

        def wait_scatter(j, slot):
            pltpu.make_async_copy(bufs[slot], out_slice(j), ssem[slot]).wait()

        start_gather(0, 0)
        start_gather(1, 1)
        for off in range(4):
            j = off
            wait_gather(j, off)
            scale(off)
            start_scatter(j, off)
            nslot = (off + 2) % 4
            if off >= 2:
                wait_scatter(j - 2, nslot)
            start_gather(j + 2, nslot)

        def group_body(g, carry):
            j0 = g * 4
            for off in range(4):
                j = j0 + off
                nslot = (off + 2) % 4
                wait_gather(j, off)
                scale(off)
                start_scatter(j, off)
                wait_scatter(j - 2, nslot)
                start_gather(j + 2, nslot)
            return carry

        lax.fori_loop(1, groups - 1, group_body, 0)

        j0 = (groups - 1) * 4
        for off in range(4):
            j = j0 + off
            nslot = (off + 2) % 4
            wait_gather(j, off)
            scale(off)
            start_scatter(j, off)
            wait_scatter(j - 2, nslot)
            if off < 2:
                start_gather(j + 2, nslot)
        wait_scatter(T - 2, 2)
        wait_scatter(T - 1, 3)

    return k


def kernel(x, lut):
    B, S = x.shape
    R = B * S
    xt = jnp.transpose(x).reshape(R // C, C)
    out_flat = _build_sc_kernel(R)(lut, xt)
    return out_flat.reshape(S, B, D_MODEL)
